# baseline (speedup 1.0000x reference)
import functools

import jax
import jax.numpy as jnp
from jax import lax
from jax.experimental import pallas as pl
from jax.experimental.pallas import tpu as pltpu
from jax.experimental.pallas import tpu_sc as plsc

VOCAB = 1_000_000
D = 64
NSEQS = 16384
SEQ = 50
SEQ_PAD = 56
IDX_PAD = 128

NC, NS = 2, 16
NW = NC * NS

NSEQ = 8
NBUF = 2

SEQS_PER_W = NSEQS // NW
NSTEPS = SEQS_PER_W // NSEQ

_mesh = plsc.VectorSubcoreMesh(
    core_axis_name="c", subcore_axis_name="s", num_cores=NC, num_subcores=NS
)


def _pad_idx_body(i_ref, o_ref):
    x = i_ref[...]
    o_ref[...] = jnp.concatenate(
        [x, jnp.zeros((x.shape[0], IDX_PAD - SEQ), jnp.int32)], axis=1
    )


_pad_idx = pl.pallas_call(
    _pad_idx_body,
    out_shape=jax.ShapeDtypeStruct((NSEQS, IDX_PAD), jnp.int32),
    grid=(16,),
    in_specs=[pl.BlockSpec((NSEQS // 16, SEQ), lambda i: (i, 0))],
    out_specs=pl.BlockSpec((NSEQS // 16, IDX_PAD), lambda i: (i, 0)),
)


@functools.partial(
    pl.kernel,
    out_type=jax.ShapeDtypeStruct((NSEQS, SEQ_PAD, D), jnp.float32),
    mesh=_mesh,
    scratch_types=[
        pltpu.VMEM((NBUF, NSEQ, IDX_PAD), jnp.int32),
        pltpu.VMEM((NBUF, NSEQ, SEQ_PAD, D), jnp.float32),
        pltpu.SemaphoreType.DMA,
        pltpu.SemaphoreType.DMA,
    ],
    compiler_params=pltpu.CompilerParams(use_tc_tiling_on_sc=False),
)
def _embed_gather(table_hbm, idx_hbm, out_hbm, idx_v, rows_v, sem0, sem1):
    sems = (sem0, sem1)
    wid = lax.axis_index("s") * NC + lax.axis_index("c")
    seq0 = wid * SEQS_PER_W

    def fire(slot, s):
        pltpu.sync_copy(idx_hbm.at[pl.ds(seq0 + s * NSEQ, NSEQ)], idx_v.at[slot])
        for j in range(NSEQ):
            pltpu.async_copy(
                table_hbm.at[idx_v.at[slot, j, pl.ds(0, SEQ_PAD)]],
                rows_v.at[slot, j],
                sems[slot],
            )

    def drain_flush(slot, s):
        pltpu.make_async_copy(
            out_hbm.at[pl.ds(0, NSEQ)],
            rows_v.at[slot],
            sems[slot],
        ).wait()
        pltpu.sync_copy(rows_v.at[slot], out_hbm.at[pl.ds(seq0 + s * NSEQ, NSEQ)])

    for b in range(NBUF):
        fire(b, b)

    @pl.loop(0, NSTEPS, step=NBUF)
    def _(g):
        for b in range(NBUF):
            s = g + b
            drain_flush(b, s)

            @pl.when(s + NBUF < NSTEPS)
            def _():
                fire(b, s + NBUF)


def kernel(token_ids, embeddings):
    idx_padded = _pad_idx(token_ids.astype(jnp.int32))
    out_padded = _embed_gather(embeddings, idx_padded)
    return out_padded[:, :SEQ, :]


# speedup vs baseline: 2.7328x; 2.7328x over previous
import functools

import jax
import jax.numpy as jnp
from jax import lax
from jax.experimental import pallas as pl
from jax.experimental.pallas import tpu as pltpu
from jax.experimental.pallas import tpu_sc as plsc

VOCAB = 1_000_000
D = 64
NSEQS = 16384
SEQ = 50
NTOK = NSEQS * SEQ
IDXW = 128
IDXROWS = NTOK // IDXW

NC, NS = 2, 16
NW = NC * NS

K = 4
NBUF = 2

ROWS_PER_W = IDXROWS // NW
NSTEPS = ROWS_PER_W // K

_mesh = plsc.VectorSubcoreMesh(
    core_axis_name="c", subcore_axis_name="s", num_cores=NC, num_subcores=NS
)


@functools.partial(
    pl.kernel,
    out_type=jax.ShapeDtypeStruct((NTOK, D), jnp.float32),
    mesh=_mesh,
    scratch_types=[
        pltpu.VMEM((NBUF, K, IDXW), jnp.int32),
        pltpu.VMEM((NBUF, K * IDXW, D), jnp.float32),
        pltpu.SemaphoreType.DMA,
        pltpu.SemaphoreType.DMA,
    ],
    compiler_params=pltpu.CompilerParams(use_tc_tiling_on_sc=False),
)
def _embed_gather(table_hbm, idx_hbm, out_hbm, idx_v, rows_v, sem0, sem1):
    sems = (sem0, sem1)
    wid = lax.axis_index("s") * NC + lax.axis_index("c")
    row0 = wid * ROWS_PER_W

    def fire(slot, s):
        pltpu.sync_copy(idx_hbm.at[pl.ds(row0 + s * K, K)], idx_v.at[slot])
        for j in range(K):
            pltpu.async_copy(
                table_hbm.at[idx_v.at[slot, j]],
                rows_v.at[slot, pl.ds(j * IDXW, IDXW)],
                sems[slot],
            )

    def drain_flush(slot, s):
        pltpu.make_async_copy(
            out_hbm.at[pl.ds(0, K * IDXW)],
            rows_v.at[slot],
            sems[slot],
        ).wait()
        pltpu.sync_copy(
            rows_v.at[slot], out_hbm.at[pl.ds((row0 + s * K) * IDXW, K * IDXW)]
        )

    for b in range(NBUF):
        fire(b, b)

    @pl.loop(0, NSTEPS, step=NBUF)
    def _(g):
        for b in range(NBUF):
            s = g + b
            drain_flush(b, s)

            @pl.when(s + NBUF < NSTEPS)
            def _():
                fire(b, s + NBUF)


def kernel(token_ids, embeddings):
    idx = token_ids.astype(jnp.int32).reshape(IDXROWS, IDXW)
    out_flat = _embed_gather(embeddings, idx)
    return out_flat.reshape(NSEQS, SEQ, D)
